# baseline (speedup 1.0000x reference)
import jax
import jax.numpy as jnp
from jax.experimental import pallas as pl
from jax.experimental.pallas import tpu as pltpu


def _make_merged_kernel(k, n, d, tile, tp, tn):
    ns = n // tile
    npc = n // tp
    nbt = n // tn
    half = tile // 2

    def body(adj0_ref, adj1_ref, h0_ref, s_ref, out_ref,
             atb_ref, degr_ref, ua_ref, ub_ref, outt_ref):
        i = pl.program_id(0)

        @pl.when(i == 0)
        def _seed():
            s_col = s_ref[...]
            for ci in range(npc):
                cols = pl.ds(ci * tp, tp)
                ft = h0_ref[pl.ds(ci * tp, tp), :].T
                ua_ref[:, cols] = ft.astype(jnp.bfloat16)
                score = jnp.sum(ft * s_col, axis=0, keepdims=True)
                outt_ref[:, cols] = jax.nn.sigmoid(score) * ft

        @pl.when(i < ns)
        def _phase1():
            for hh, aref in enumerate((adj0_ref, adj1_ref)):
                t = aref[...]
                acc = jnp.zeros((1, half), jnp.float32)
                for c in range(npc):
                    ttf = t[:, c * tp:(c + 1) * tp].T
                    atb_ref[pl.ds(c * tp, tp),
                            pl.ds(i * tile + hh * half, half)] = (
                        ttf.astype(jnp.bfloat16))
                    acc = acc + jnp.sum(ttf, axis=0, keepdims=True)
                degr_ref[:, pl.ds(i * tile + hh * half, half)] = acc

        @pl.when(i == ns)
        def _phase2():
            s_col = s_ref[...]

            for ci in range(nbt):
                cols = pl.ds(ci * tn, tn)
                nr = jax.lax.rsqrt(degr_ref[:, cols])
                ua_ref[:, cols] = (ua_ref[:, cols].astype(jnp.float32)
                                   * nr).astype(jnp.bfloat16)

            bufs = (ua_ref, ub_ref)
            for t in range(k):
                src = bufs[t % 2]
                dst = bufs[(t + 1) % 2]
                for ci in range(nbt):
                    cols = pl.ds(ci * tn, tn)
                    nr = jax.lax.rsqrt(degr_ref[:, cols])
                    y = jnp.dot(src[...], atb_ref[:, cols],
                                preferred_element_type=jnp.float32)
                    h = y * nr
                    score = jnp.sum(h * s_col, axis=0, keepdims=True)
                    outt_ref[:, cols] = (outt_ref[:, cols]
                                         + jax.nn.sigmoid(score) * h)
                    if t < k - 1:
                        dst[:, cols] = (h * nr).astype(jnp.bfloat16)

            for ci in range(npc):
                cols = pl.ds(ci * tp, tp)
                out_ref[pl.ds(ci * tp, tp), :] = outt_ref[:, cols].T

    return body


def _dagnn(adj, feats, s, k, tile=512, tp=512, tn=512):
    n, d = feats.shape
    tile = min(tile, n)
    tp = min(tp, n)
    tn = min(tn, n)
    ns = n // tile
    return pl.pallas_call(
        _make_merged_kernel(k, n, d, tile, tp, tn),
        out_shape=jax.ShapeDtypeStruct((n, d), jnp.float32),
        grid=(ns + 1,),
        in_specs=[
            pl.BlockSpec((tile // 2, n),
                         lambda i: (2 * jnp.minimum(i, ns - 1), 0)),
            pl.BlockSpec((tile // 2, n),
                         lambda i: (2 * jnp.minimum(i, ns - 1) + 1, 0)),
            pl.BlockSpec((n, d), lambda i: (0, 0)),
            pl.BlockSpec((d, 1), lambda i: (0, 0)),
        ],
        out_specs=pl.BlockSpec((n, d), lambda i: (0, 0)),
        scratch_shapes=[
            pltpu.VMEM((n, n), jnp.bfloat16),
            pltpu.VMEM((1, n), jnp.float32),
            pltpu.VMEM((d, n), jnp.bfloat16),
            pltpu.VMEM((d, n), jnp.bfloat16),
            pltpu.VMEM((d, n), jnp.float32),
        ],
        compiler_params=pltpu.CompilerParams(
            dimension_semantics=("arbitrary",),
            vmem_limit_bytes=63 * 1024 * 1024),
        cost_estimate=pl.CostEstimate(
            flops=2 * k * n * n * d,
            transcendentals=(k + 1) * n,
            bytes_accessed=4 * n * n + 4 * 3 * n * d),
    )(adj.astype(jnp.float32), adj.astype(jnp.float32),
      feats.astype(jnp.float32), s.astype(jnp.float32))


def kernel(adj, feats, s):
    return _dagnn(adj, feats, s, 4)


# speedup vs baseline: 1.1117x; 1.1117x over previous
import jax
import jax.numpy as jnp
from jax.experimental import pallas as pl
from jax.experimental.pallas import tpu as pltpu


def _make_merged_kernel(k, n, d, tile, tn):
    ns = n // tile
    nb = n // tn

    half = tile // 2

    def body(adj0_ref, adj1_ref, h0_ref, s_ref, out_ref,
             atb_ref, degr_ref, h_ref, u_ref, outt_ref):
        i = pl.program_id(0)

        @pl.when(i < ns)
        def _phase1():
            for hh, aref in enumerate((adj0_ref, adj1_ref)):
                t = aref[...]
                acc = jnp.zeros((1, half), jnp.float32)
                for c in range(nb):
                    ttb = t[:, c * tn:(c + 1) * tn].astype(jnp.bfloat16).T
                    atb_ref[pl.ds(c * tn, tn),
                            pl.ds(i * tile + hh * half, half)] = ttb
                    acc = acc + jnp.sum(ttb, axis=0, keepdims=True,
                                        dtype=jnp.float32)
                degr_ref[:, pl.ds(i * tile + hh * half, half)] = acc

        @pl.when(i == ns)
        def _phase2():
            s_col = s_ref[...]

            for ci in range(nb):
                cols = pl.ds(ci * tn, tn)
                ft = h0_ref[pl.ds(ci * tn, tn), :].T
                h_ref[:, cols] = ft
                score = jnp.sum(ft * s_col, axis=0, keepdims=True)
                outt_ref[:, cols] = jax.nn.sigmoid(score) * ft

            for _ in range(k):
                for ci in range(nb):
                    cols = pl.ds(ci * tn, tn)
                    nr = jax.lax.rsqrt(degr_ref[:, cols])
                    u_ref[:, cols] = (h_ref[:, cols] * nr).astype(jnp.bfloat16)
                for ci in range(nb):
                    cols = pl.ds(ci * tn, tn)
                    nr = jax.lax.rsqrt(degr_ref[:, cols])
                    y = jnp.dot(u_ref[...], atb_ref[:, cols],
                                preferred_element_type=jnp.float32)
                    h = y * nr
                    h_ref[:, cols] = h
                    score = jnp.sum(h * s_col, axis=0, keepdims=True)
                    outt_ref[:, cols] = (outt_ref[:, cols]
                                         + jax.nn.sigmoid(score) * h)

            for ci in range(nb):
                out_ref[pl.ds(ci * tn, tn), :] = outt_ref[:, pl.ds(ci * tn, tn)].T

    return body


def _dagnn(adj, feats, s, k, tile=512, tn=512):
    n, d = feats.shape
    tile = min(tile, n)
    tn = min(tn, n)
    ns = n // tile
    return pl.pallas_call(
        _make_merged_kernel(k, n, d, tile, tn),
        out_shape=jax.ShapeDtypeStruct((n, d), jnp.float32),
        grid=(ns + 1,),
        in_specs=[
            pl.BlockSpec((tile // 2, n),
                         lambda i: (2 * jnp.minimum(i, ns - 1), 0)),
            pl.BlockSpec((tile // 2, n),
                         lambda i: (2 * jnp.minimum(i, ns - 1) + 1, 0)),
            pl.BlockSpec((n, d), lambda i: (0, 0)),
            pl.BlockSpec((d, 1), lambda i: (0, 0)),
        ],
        out_specs=pl.BlockSpec((n, d), lambda i: (0, 0)),
        scratch_shapes=[
            pltpu.VMEM((n, n), jnp.bfloat16),
            pltpu.VMEM((1, n), jnp.float32),
            pltpu.VMEM((d, n), jnp.float32),
            pltpu.VMEM((d, n), jnp.bfloat16),
            pltpu.VMEM((d, n), jnp.float32),
        ],
        compiler_params=pltpu.CompilerParams(
            dimension_semantics=("arbitrary",),
            vmem_limit_bytes=63 * 1024 * 1024),
        cost_estimate=pl.CostEstimate(
            flops=2 * k * n * n * d,
            transcendentals=(k + 1) * n,
            bytes_accessed=4 * n * n + 4 * 3 * n * d),
    )(adj.astype(jnp.float32), adj.astype(jnp.float32),
      feats.astype(jnp.float32), s.astype(jnp.float32))


def kernel(adj, feats, s):
    return _dagnn(adj, feats, s, 4)
